# Initial kernel scaffold; baseline (speedup 1.0000x reference)
import jax
import jax.numpy as jnp
from jax.experimental import pallas as pl


def kernel(ref_points_f, src_points_f, ref_points_c, src_points_c, ref_feats_c, src_feats_c, ref_feats_f, src_feats_f, ref_m_scores, src_m_scores):
    raise NotImplementedError("write your pallas kernel here")


# speedup vs baseline: 1.5233x; 1.5233x over previous
import functools
import math

import jax
import jax.numpy as jnp
from jax.experimental import pallas as pl
from jax.experimental.pallas import tpu as pltpu

NF = 20000
NC = 512
DF = 128
K = 64
C = 256
NF_PAD = NF + 8
AUG = 136
INF = jnp.inf

_PT = 2000


def _partition_body(pts_ref, nodes_ref, nid_ref, nd_ref):
    pts = pts_ref[...]
    nodes = nodes_ref[...]
    d = jnp.dot(pts, nodes, preferred_element_type=jnp.float32)
    pp = jnp.sum(pts * pts, axis=1)
    cc = jnp.sum(nodes * nodes, axis=0)
    dist = jnp.maximum(pp[:, None] + cc[None, :] - 2.0 * d, 0.0)
    m = jnp.min(dist, axis=1)
    lane = jax.lax.broadcasted_iota(jnp.int32, dist.shape, 1)
    am = jnp.min(jnp.where(dist == m[:, None], lane, NC), axis=1)
    nid_ref[0, 0, :] = am
    nd_ref[0, 0, :] = m


def _partition(points_f, points_c):
    pts8 = jnp.pad(points_f, ((0, 0), (0, 5)))
    nodes8 = jnp.pad(points_c, ((0, 0), (0, 5))).T
    grid = NF // _PT
    nid, nd = pl.pallas_call(
        _partition_body,
        grid=(grid,),
        in_specs=[
            pl.BlockSpec((_PT, 8), lambda i: (i, 0)),
            pl.BlockSpec((8, NC), lambda i: (0, 0)),
        ],
        out_specs=[
            pl.BlockSpec((1, 1, _PT), lambda i: (i, 0, 0)),
            pl.BlockSpec((1, 1, _PT), lambda i: (i, 0, 0)),
        ],
        out_shape=[
            jax.ShapeDtypeStruct((grid, 1, _PT), jnp.int32),
            jax.ShapeDtypeStruct((grid, 1, _PT), jnp.float32),
        ],
    )(pts8, nodes8)
    return nid.reshape(1, NF), nd.reshape(1, NF)


_NB = 64


def _select_body(nid_ref, nd_ref, idx_ref, cnt_ref, m_ref):
    blk = pl.program_id(0)
    rows = blk * _NB + jax.lax.broadcasted_iota(jnp.int32, (_NB, NF), 0)
    M = jnp.where(nid_ref[...] == rows, nd_ref[...], INF)
    m_ref[...] = M
    cnt_ref[0, 0, :] = jnp.sum((M < INF).astype(jnp.int32), axis=1)

    def step(j, _):
        M = m_ref[...]
        m = jnp.min(M, axis=1)
        lane = jax.lax.broadcasted_iota(jnp.int32, (_NB, NF), 1)
        am = jnp.min(jnp.where(M == m[:, None], lane, NF), axis=1)
        am = jnp.where(m < INF, am, NF)
        m_ref[...] = jnp.where(lane == am[:, None], INF, M)
        col = jax.lax.broadcasted_iota(jnp.int32, (_NB, K), 1)
        idx_ref[...] = jnp.where(col == j, am[:, None], idx_ref[...])
        return 0

    jax.lax.fori_loop(0, K, step, 0)


def _select(nid, nd):
    grid = NC // _NB
    idx, cnt = pl.pallas_call(
        _select_body,
        grid=(grid,),
        in_specs=[
            pl.BlockSpec((1, NF), lambda i: (0, 0)),
            pl.BlockSpec((1, NF), lambda i: (0, 0)),
        ],
        out_specs=[
            pl.BlockSpec((_NB, K), lambda i: (i, 0)),
            pl.BlockSpec((1, 1, _NB), lambda i: (i, 0, 0)),
        ],
        out_shape=[
            jax.ShapeDtypeStruct((NC, K), jnp.int32),
            jax.ShapeDtypeStruct((grid, 1, _NB), jnp.int32),
        ],
        scratch_shapes=[pltpu.VMEM((_NB, NF), jnp.float32)],
    )(nid, nd)
    return idx, cnt.reshape(NC)


def _coarse_body(rf_ref, sf_ref, rcnt_ref, scnt_ref, rout_ref, sout_ref,
                 gout_ref, s_ref):
    rf = rf_ref[...]
    sf = sf_ref[...]
    rn = rf / (jnp.sqrt(jnp.sum(rf * rf, axis=1, keepdims=True)) + 1e-8)
    sn = sf / (jnp.sqrt(jnp.sum(sf * sf, axis=1, keepdims=True)) + 1e-8)
    sim = jnp.exp(jax.lax.dot_general(rn, sn, (((1,), (1,)), ((), ())),
                                      preferred_element_type=jnp.float32))
    rsum = jnp.sum(sim, axis=1, keepdims=True)
    csum = jnp.sum(sim, axis=0, keepdims=True)
    scores = (sim / rsum) * (sim / csum)
    mask = (rcnt_ref[...] > 0) & (scnt_ref[...] > 0)
    s_ref[...] = jnp.where(mask, scores, 0.0)

    rows2 = jax.lax.broadcasted_iota(jnp.int32, (NC, NC), 0)
    lanes2 = jax.lax.broadcasted_iota(jnp.int32, (NC, NC), 1)
    lane_c = jax.lax.broadcasted_iota(jnp.int32, (1, C), 1)

    def step(t, _):
        S = s_ref[...]
        g = jnp.max(S, axis=(0, 1), keepdims=True)
        rm = jnp.max(S, axis=1, keepdims=True)
        rsel = jnp.min(jnp.where(rm == g, rows2[:, :1], NC),
                       axis=(0, 1), keepdims=True)
        sel_row = rows2 == rsel
        csel = jnp.min(jnp.where(sel_row & (S == g), lanes2, NC),
                       axis=(0, 1), keepdims=True)
        s_ref[...] = jnp.where(sel_row & (lanes2 == csel), -1.0, S)
        hit = lane_c == t
        rout_ref[...] = jnp.where(hit, rsel, rout_ref[...])
        sout_ref[...] = jnp.where(hit, csel, sout_ref[...])
        gout_ref[...] = jnp.where(hit, g, gout_ref[...])
        return 0

    jax.lax.fori_loop(0, C, step, 0)


def _coarse(ref_feats_c, src_feats_c, rcnt, scnt):
    rout, sout, gout = pl.pallas_call(
        _coarse_body,
        in_specs=[
            pl.BlockSpec(ref_feats_c.shape, lambda: (0, 0)),
            pl.BlockSpec(src_feats_c.shape, lambda: (0, 0)),
            pl.BlockSpec((NC, 1), lambda: (0, 0)),
            pl.BlockSpec((1, NC), lambda: (0, 0)),
        ],
        out_specs=[
            pl.BlockSpec((1, C), lambda: (0, 0)),
            pl.BlockSpec((1, C), lambda: (0, 0)),
            pl.BlockSpec((1, C), lambda: (0, 0)),
        ],
        out_shape=[
            jax.ShapeDtypeStruct((1, C), jnp.int32),
            jax.ShapeDtypeStruct((1, C), jnp.int32),
            jax.ShapeDtypeStruct((1, C), jnp.float32),
        ],
        scratch_shapes=[pltpu.VMEM((NC, NC), jnp.float32)],
    )(ref_feats_c, src_feats_c, rcnt.reshape(NC, 1), scnt.reshape(1, NC))
    return rout.reshape(C), sout.reshape(C), gout.reshape(C)


_EYE_SCALE = 1.0 / math.sqrt(float(DF))


def _fine_body(rc_ref, sc_ref, ridx_ref, sidx_ref, rtab_ref, stab_ref,
               out_ref, rfb_ref, sfb_ref):
    for j in range(K):
        ri = ridx_ref[0, 0, j]
        rfb_ref[pl.ds(j, 1), :] = rtab_ref[pl.ds(ri, 1), :]
        si = sidx_ref[0, 0, j]
        sfb_ref[pl.ds(j, 1), :] = stab_ref[pl.ds(si, 1), :]

    rfb = rfb_ref[...]
    sfb = sfb_ref[...]
    rfeat = rfb[:, :DF]
    sfeat = sfb[:, :DF]
    rext = rfb[:, DF:DF + 8]
    sext = sfb[:, DF:DF + 8]
    rscore_c = rext[:, 0:1]
    rvalid_c = rext[:, 1:2] > 0.5
    sscore_c = sext[:, 0:1]
    svalid_c = sext[:, 1:2] > 0.5

    eye = (jax.lax.broadcasted_iota(jnp.int32, (K, K), 0)
           == jax.lax.broadcasted_iota(jnp.int32, (K, K), 1))
    sscore_r = jnp.sum(jnp.where(eye, sscore_c, 0.0), axis=0, keepdims=True)
    svalid_r = jnp.sum(jnp.where(eye & svalid_c, 1.0, 0.0), axis=0,
                       keepdims=True) > 0.5

    sim = jax.lax.dot_general(rfeat, sfeat, (((1,), (1,)), ((), ())),
                              preferred_element_type=jnp.float32) * _EYE_SCALE
    mask = rvalid_c & svalid_r
    sim = jnp.where(mask, sim, -1e9)
    m2 = jnp.max(sim, axis=1, keepdims=True)
    e2 = jnp.exp(sim - m2)
    s2 = e2 / jnp.sum(e2, axis=1, keepdims=True)
    m1 = jnp.max(sim, axis=0, keepdims=True)
    e1 = jnp.exp(sim - m1)
    s1 = e1 / jnp.sum(e1, axis=0, keepdims=True)
    scores = s2 * s1 * rscore_c * sscore_r
    out_ref[0] = jnp.where(mask, scores, 0.0)


def _fine(ref_corr, src_corr, ref_knn_idx, src_knn_idx, rtab, stab):
    grid_spec = pltpu.PrefetchScalarGridSpec(
        num_scalar_prefetch=2,
        grid=(C,),
        in_specs=[
            pl.BlockSpec((1, 1, K), lambda i, rc, sc: (rc[i], 0, 0),
                         memory_space=pltpu.SMEM),
            pl.BlockSpec((1, 1, K), lambda i, rc, sc: (sc[i], 0, 0),
                         memory_space=pltpu.SMEM),
            pl.BlockSpec((NF_PAD, AUG), lambda i, rc, sc: (0, 0)),
            pl.BlockSpec((NF_PAD, AUG), lambda i, rc, sc: (0, 0)),
        ],
        out_specs=pl.BlockSpec((1, K, K), lambda i, rc, sc: (i, 0, 0)),
        scratch_shapes=[
            pltpu.VMEM((K, AUG), jnp.float32),
            pltpu.VMEM((K, AUG), jnp.float32),
        ],
    )
    return pl.pallas_call(
        _fine_body,
        grid_spec=grid_spec,
        out_shape=jax.ShapeDtypeStruct((C, K, K), jnp.float32),
    )(ref_corr, src_corr, ref_knn_idx.reshape(NC, 1, K),
      src_knn_idx.reshape(NC, 1, K), rtab, stab)


def _aug_table(feats_f, m_scores):
    tab = jnp.zeros((NF_PAD, AUG), jnp.float32)
    tab = tab.at[:NF, :DF].set(feats_f)
    tab = tab.at[:NF, DF].set(m_scores)
    tab = tab.at[:NF, DF + 1].set(1.0)
    return tab


def kernel(ref_points_f, src_points_f, ref_points_c, src_points_c,
           ref_feats_c, src_feats_c, ref_feats_f, src_feats_f,
           ref_m_scores, src_m_scores):
    r_nid, r_nd = _partition(ref_points_f, ref_points_c)
    s_nid, s_nd = _partition(src_points_f, src_points_c)

    r_knn_idx, r_cnt = _select(r_nid, r_nd)
    s_knn_idx, s_cnt = _select(s_nid, s_nd)

    ref_corr, src_corr, corr_scores = _coarse(
        ref_feats_c, src_feats_c, r_cnt, s_cnt)

    rtab = _aug_table(ref_feats_f, ref_m_scores)
    stab = _aug_table(src_feats_f, src_m_scores)

    matching = _fine(ref_corr, src_corr, r_knn_idx, s_knn_idx, rtab, stab)
    return ref_corr, src_corr, corr_scores, matching
